# baseline (speedup 1.0000x reference)
import functools

import jax
import jax.numpy as jnp
from jax import lax
from jax.experimental import pallas as pl
from jax.experimental.pallas import tpu as pltpu
from jax.experimental.pallas import tpu_sc as plsc

N_NODES = 10000
N_EDGES = 160000
N_ORB = 37
MAX_Z = 100
EMBED = 16
OUT_W = N_ORB * EMBED
KDIM = 128


_NC = 2
_NS = 16
_NW = _NC * _NS
_EPW = N_EDGES // _NW
_EPW_PAD = ((_EPW + 15) // 16) * 16


def _make_zj_kernel():
    mesh = plsc.VectorSubcoreMesh(core_axis_name="c", subcore_axis_name="s")

    @functools.partial(
        pl.kernel,
        out_type=jax.ShapeDtypeStruct((N_EDGES,), jnp.int32),
        mesh=mesh,
        scratch_types=[
            pltpu.VMEM((N_NODES,), jnp.int32),
            pltpu.VMEM((_EPW_PAD,), jnp.int32),
            pltpu.VMEM((_EPW_PAD,), jnp.int32),
        ],
        compiler_params=pltpu.CompilerParams(needs_layout_passes=False),
    )
    def zj_kernel(z_hbm, idx_hbm, out_hbm, z_v, idx_v, zj_v):
        wid = lax.axis_index("s") * _NC + lax.axis_index("c")
        base = wid * _EPW
        pltpu.sync_copy(z_hbm, z_v)
        idx_v[pl.ds(_EPW_PAD - 16, 16)] = jnp.zeros((16,), jnp.int32)
        pltpu.sync_copy(idx_hbm.at[pl.ds(base, _EPW)], idx_v.at[pl.ds(0, _EPW)])

        def body(i, carry):
            idx16 = idx_v[pl.ds(i * 16, 16)]
            zj_v[pl.ds(i * 16, 16)] = plsc.load_gather(z_v, [idx16])
            return carry

        lax.fori_loop(0, _EPW_PAD // 16, body, 0)
        pltpu.sync_copy(zj_v.at[pl.ds(0, _EPW)], out_hbm.at[pl.ds(base, _EPW)])

    return zj_kernel


_B = 6400
_NB = N_EDGES // _B


def _expand_body(zj_ref, vet_ref, out_ref):
    zj_row = zj_ref[0]
    iota = lax.broadcasted_iota(jnp.int32, (KDIM, _B), 0)
    onehot_t = (iota == zj_row).astype(jnp.bfloat16)
    out_ref[...] = lax.dot_general(
        vet_ref[...], onehot_t,
        dimension_numbers=(((1,), (0,)), ((), ())),
        preferred_element_type=jnp.float32,
    )


def _expand_call(zj3, vet):
    return pl.pallas_call(
        _expand_body,
        grid=(_NB,),
        in_specs=[
            pl.BlockSpec((1, 1, _B), lambda i: (i, 0, 0)),
            pl.BlockSpec((OUT_W, KDIM), lambda i: (0, 0)),
        ],
        out_specs=pl.BlockSpec((OUT_W, _B), lambda i: (0, i)),
        out_shape=jax.ShapeDtypeStruct((OUT_W, N_EDGES), jnp.float32),
    )(zj3, vet)


def kernel(z, idx_j, valence):
    z = z.astype(jnp.int32)
    idx_j = idx_j.astype(jnp.int32)
    zj = _make_zj_kernel()(z, idx_j)
    ve = jnp.repeat(valence, EMBED, axis=1)
    ve = jnp.pad(ve, ((0, KDIM - MAX_Z), (0, 0)))
    vet = ve.T.astype(jnp.bfloat16)
    out_t = _expand_call(zj.reshape(_NB, 1, _B), vet)
    return out_t.reshape(N_ORB, EMBED, N_EDGES).transpose(2, 0, 1)


# speedup vs baseline: 1.0100x; 1.0100x over previous
import functools

import jax
import jax.numpy as jnp
from jax import lax
from jax.experimental import pallas as pl
from jax.experimental.pallas import tpu as pltpu
from jax.experimental.pallas import tpu_sc as plsc

N_NODES = 10000
N_EDGES = 160000
N_ORB = 37
MAX_Z = 100
EMBED = 16
OUT_W = N_ORB * EMBED
KDIM = 128


_NC = 2
_NS = 16
_NW = _NC * _NS
_EPW = N_EDGES // _NW
_EPW_PAD = ((_EPW + 15) // 16) * 16


def _make_zj_kernel():
    mesh = plsc.VectorSubcoreMesh(core_axis_name="c", subcore_axis_name="s")

    @functools.partial(
        pl.kernel,
        out_type=jax.ShapeDtypeStruct((N_EDGES,), jnp.int32),
        mesh=mesh,
        scratch_types=[
            pltpu.VMEM((N_NODES,), jnp.int32),
            pltpu.VMEM((_EPW_PAD,), jnp.int32),
            pltpu.VMEM((_EPW_PAD,), jnp.int32),
        ],
        compiler_params=pltpu.CompilerParams(needs_layout_passes=False),
    )
    def zj_kernel(z_hbm, idx_hbm, out_hbm, z_v, idx_v, zj_v):
        wid = lax.axis_index("s") * _NC + lax.axis_index("c")
        base = wid * _EPW
        pltpu.sync_copy(z_hbm, z_v)
        idx_v[pl.ds(_EPW_PAD - 16, 16)] = jnp.zeros((16,), jnp.int32)
        pltpu.sync_copy(idx_hbm.at[pl.ds(base, _EPW)], idx_v.at[pl.ds(0, _EPW)])

        def body(i, carry):
            idx16 = idx_v[pl.ds(i * 16, 16)]
            zj_v[pl.ds(i * 16, 16)] = plsc.load_gather(z_v, [idx16])
            return carry

        lax.fori_loop(0, _EPW_PAD // 16, body, 0)
        pltpu.sync_copy(zj_v.at[pl.ds(0, _EPW)], out_hbm.at[pl.ds(base, _EPW)])

    return zj_kernel


_B = 3200
_NB = N_EDGES // _B


def _expand_body(zj_ref, vet_ref, out_ref):
    zj_row = zj_ref[0]
    iota = lax.broadcasted_iota(jnp.int32, (KDIM, _B), 0)
    onehot_t = (iota == zj_row).astype(jnp.bfloat16)
    out_ref[...] = lax.dot_general(
        vet_ref[...], onehot_t,
        dimension_numbers=(((1,), (0,)), ((), ())),
        preferred_element_type=jnp.float32,
    )


def _expand_call(zj3, vet):
    return pl.pallas_call(
        _expand_body,
        grid=(_NB,),
        in_specs=[
            pl.BlockSpec((1, 1, _B), lambda i: (i, 0, 0)),
            pl.BlockSpec((OUT_W, KDIM), lambda i: (0, 0)),
        ],
        out_specs=pl.BlockSpec((OUT_W, _B), lambda i: (0, i)),
        out_shape=jax.ShapeDtypeStruct((OUT_W, N_EDGES), jnp.float32),
    )(zj3, vet)


def kernel(z, idx_j, valence):
    z = z.astype(jnp.int32)
    idx_j = idx_j.astype(jnp.int32)
    zj = _make_zj_kernel()(z, idx_j)
    ve = jnp.repeat(valence, EMBED, axis=1)
    ve = jnp.pad(ve, ((0, KDIM - MAX_Z), (0, 0)))
    vet = ve.T.astype(jnp.bfloat16)
    out_t = _expand_call(zj.reshape(_NB, 1, _B), vet)
    return out_t.reshape(N_ORB, EMBED, N_EDGES).transpose(2, 0, 1)
